# baseline (speedup 1.0000x reference)
import functools

import jax
import jax.numpy as jnp
from jax import lax
from jax.experimental import pallas as pl
from jax.experimental.pallas import tpu as pltpu
from jax.experimental.pallas import tpu_sc as plsc


def _dynmap_sc(pos_h, pos_t, pos_r, neg_h, neg_t, neg_r,
               ent_emb, rel_emb, ent_proj, rel_proj):
    B = pos_h.shape[0]
    D = ent_emb.shape[1]
    info = plsc.get_sparse_core_info()
    NC, NS, L = info.num_cores, info.num_subcores, info.num_lanes
    NW = NC * NS
    C = 128
    b_per_w = B // NW
    n_chunks = b_per_w // C
    KV = D // L

    mesh = plsc.VectorSubcoreMesh(core_axis_name="c", subcore_axis_name="s")
    f32 = jnp.float32
    out_type = (
        jax.ShapeDtypeStruct((B,), f32),
        jax.ShapeDtypeStruct((B,), f32),
        jax.ShapeDtypeStruct((B, D), f32),
        jax.ShapeDtypeStruct((B, D), f32),
        jax.ShapeDtypeStruct((B, D), f32),
        jax.ShapeDtypeStruct((B, D), f32),
    )

    @functools.partial(
        pl.kernel,
        mesh=mesh,
        out_type=out_type,
        compiler_params=pltpu.CompilerParams(
            needs_layout_passes=False, use_tc_tiling_on_sc=False),
        scratch_types=[
            pltpu.VMEM((C,), jnp.int32),
            pltpu.VMEM((C,), jnp.int32),
            pltpu.VMEM((C,), jnp.int32),
            pltpu.VMEM((C, D), f32),
            pltpu.VMEM((C, D), f32),
            pltpu.VMEM((C, D), f32),
            pltpu.VMEM((C, D), f32),
            pltpu.VMEM((C, D), f32),
            pltpu.VMEM((C, D), f32),
            pltpu.VMEM((C,), f32),
            pltpu.SemaphoreType.DMA,
        ],
    )
    def k(ph, pt, pr, nh, nt, nr, ee, rel_e, ep, rel_p,
          pos_o, neg_o, phe_o, pte_o, nhe_o, nte_o,
          hi, ti, ri, he, hp, te, tp, re_v, rp_v, dv, sem):
        wid = lax.axis_index("s") * NC + lax.axis_index("c")
        lane = lax.iota(jnp.int32, L)

        def do_side(h_idx, t_idx, r_idx, dist_o, h_out, t_out):
            def chunk_body(c, carry):
                base = wid * b_per_w + c * C
                pltpu.sync_copy(h_idx.at[pl.ds(base, C)], hi)
                pltpu.sync_copy(t_idx.at[pl.ds(base, C)], ti)
                pltpu.sync_copy(r_idx.at[pl.ds(base, C)], ri)
                cps = [
                    pltpu.async_copy(ee.at[hi], he, sem),
                    pltpu.async_copy(ep.at[hi], hp, sem),
                    pltpu.async_copy(ee.at[ti], te, sem),
                    pltpu.async_copy(ep.at[ti], tp, sem),
                    pltpu.async_copy(rel_e.at[ri], re_v, sem),
                    pltpu.async_copy(rel_p.at[ri], rp_v, sem),
                ]
                for cp in cps:
                    cp.wait()

                def group_body(g, gcarry):
                    d_vec = jnp.zeros((L,), f32)
                    for rl in range(L):
                        row = g * L + rl
                        hv = [he[row, pl.ds(kk * L, L)] for kk in range(KV)]
                        hpv = [hp[row, pl.ds(kk * L, L)] for kk in range(KV)]
                        tv = [te[row, pl.ds(kk * L, L)] for kk in range(KV)]
                        tpv = [tp[row, pl.ds(kk * L, L)] for kk in range(KV)]
                        rv = [re_v[row, pl.ds(kk * L, L)] for kk in range(KV)]
                        rpv = [rp_v[row, pl.ds(kk * L, L)] for kk in range(KV)]
                        hdot = hv[0] * hpv[0]
                        tdot = tv[0] * tpv[0]
                        for kk in range(1, KV):
                            hdot = hdot + hv[kk] * hpv[kk]
                            tdot = tdot + tv[kk] * tpv[kk]
                        sh = jnp.sum(hdot)
                        st = jnp.sum(tdot)
                        dacc = None
                        for kk in range(KV):
                            hprime = hv[kk] + sh * rpv[kk]
                            tprime = tv[kk] + st * rpv[kk]
                            he[row, pl.ds(kk * L, L)] = hprime
                            te[row, pl.ds(kk * L, L)] = tprime
                            term = jnp.abs(hprime + rv[kk] - tprime)
                            dacc = term if dacc is None else dacc + term
                        d = jnp.sum(dacc)
                        d_vec = jnp.where(lane == rl, d, d_vec)
                    dv[pl.ds(g * L, L)] = d_vec
                    return gcarry

                lax.fori_loop(0, C // L, group_body, 0)
                pltpu.sync_copy(he, h_out.at[pl.ds(base, C)])
                pltpu.sync_copy(te, t_out.at[pl.ds(base, C)])
                pltpu.sync_copy(dv, dist_o.at[pl.ds(base, C)])
                return carry

            lax.fori_loop(0, n_chunks, chunk_body, 0)

        do_side(ph, pt, pr, pos_o, phe_o, pte_o)
        do_side(nh, nt, nr, neg_o, nhe_o, nte_o)

    return k(pos_h, pos_t, pos_r, neg_h, neg_t, neg_r,
             ent_emb, rel_emb, ent_proj, rel_proj)


def kernel(pos_h, pos_t, pos_r, neg_h, neg_t, neg_r,
           ent_emb, rel_emb, ent_proj, rel_proj):
    i32 = jnp.int32
    return _dynmap_sc(pos_h.astype(i32), pos_t.astype(i32), pos_r.astype(i32),
                      neg_h.astype(i32), neg_t.astype(i32), neg_r.astype(i32),
                      ent_emb, rel_emb, ent_proj, rel_proj)


# speedup vs baseline: 1.0568x; 1.0568x over previous
import functools

import jax
import jax.numpy as jnp
from jax import lax
from jax.experimental import pallas as pl
from jax.experimental.pallas import tpu as pltpu
from jax.experimental.pallas import tpu_sc as plsc


def _dynmap_sc(pos_h, pos_t, pos_r, neg_h, neg_t, neg_r,
               ent_emb, rel_emb, ent_proj, rel_proj, d_real):
    B = pos_h.shape[0]
    D = ent_emb.shape[1]
    info = plsc.get_sparse_core_info()
    NC, NS, L = info.num_cores, info.num_subcores, info.num_lanes
    NW = NC * NS
    C = 128
    b_per_w = B // NW
    n_chunks = b_per_w // C
    KV = d_real // L

    mesh = plsc.VectorSubcoreMesh(core_axis_name="c", subcore_axis_name="s")
    f32 = jnp.float32
    out_type = (
        jax.ShapeDtypeStruct((B,), f32),
        jax.ShapeDtypeStruct((B,), f32),
        jax.ShapeDtypeStruct((B, D), f32),
        jax.ShapeDtypeStruct((B, D), f32),
        jax.ShapeDtypeStruct((B, D), f32),
        jax.ShapeDtypeStruct((B, D), f32),
    )

    @functools.partial(
        pl.kernel,
        mesh=mesh,
        out_type=out_type,
        compiler_params=pltpu.CompilerParams(needs_layout_passes=False),
        scratch_types=[
            pltpu.VMEM((C,), jnp.int32),
            pltpu.VMEM((C,), jnp.int32),
            pltpu.VMEM((C,), jnp.int32),
            pltpu.VMEM((C, D), f32),
            pltpu.VMEM((C, D), f32),
            pltpu.VMEM((C, D), f32),
            pltpu.VMEM((C, D), f32),
            pltpu.VMEM((C, D), f32),
            pltpu.VMEM((C, D), f32),
            pltpu.VMEM((C,), f32),
            pltpu.SemaphoreType.DMA,
        ],
    )
    def k(ph, pt, pr, nh, nt, nr, ee, rel_e, ep, rel_p,
          pos_o, neg_o, phe_o, pte_o, nhe_o, nte_o,
          hi, ti, ri, he, hp, te, tp, re_v, rp_v, dv, sem):
        wid = lax.axis_index("s") * NC + lax.axis_index("c")
        lane = lax.iota(jnp.int32, L)

        def do_side(h_idx, t_idx, r_idx, dist_o, h_out, t_out):
            def chunk_body(c, carry):
                base = wid * b_per_w + c * C
                pltpu.sync_copy(h_idx.at[pl.ds(base, C)], hi)
                pltpu.sync_copy(t_idx.at[pl.ds(base, C)], ti)
                pltpu.sync_copy(r_idx.at[pl.ds(base, C)], ri)
                cps = [
                    pltpu.async_copy(ee.at[hi], he, sem),
                    pltpu.async_copy(ep.at[hi], hp, sem),
                    pltpu.async_copy(ee.at[ti], te, sem),
                    pltpu.async_copy(ep.at[ti], tp, sem),
                    pltpu.async_copy(rel_e.at[ri], re_v, sem),
                    pltpu.async_copy(rel_p.at[ri], rp_v, sem),
                ]
                for cp in cps:
                    cp.wait()

                def group_body(g, gcarry):
                    d_vec = jnp.zeros((L,), f32)
                    for rl in range(L):
                        row = g * L + rl
                        hv = [he[row, pl.ds(kk * L, L)] for kk in range(KV)]
                        hpv = [hp[row, pl.ds(kk * L, L)] for kk in range(KV)]
                        tv = [te[row, pl.ds(kk * L, L)] for kk in range(KV)]
                        tpv = [tp[row, pl.ds(kk * L, L)] for kk in range(KV)]
                        rv = [re_v[row, pl.ds(kk * L, L)] for kk in range(KV)]
                        rpv = [rp_v[row, pl.ds(kk * L, L)] for kk in range(KV)]
                        hdot = hv[0] * hpv[0]
                        tdot = tv[0] * tpv[0]
                        for kk in range(1, KV):
                            hdot = hdot + hv[kk] * hpv[kk]
                            tdot = tdot + tv[kk] * tpv[kk]
                        sh = jnp.sum(hdot)
                        st = jnp.sum(tdot)
                        dacc = None
                        for kk in range(KV):
                            hprime = hv[kk] + sh * rpv[kk]
                            tprime = tv[kk] + st * rpv[kk]
                            he[row, pl.ds(kk * L, L)] = hprime
                            te[row, pl.ds(kk * L, L)] = tprime
                            term = jnp.abs(hprime + rv[kk] - tprime)
                            dacc = term if dacc is None else dacc + term
                        d = jnp.sum(dacc)
                        d_vec = jnp.where(lane == rl, d, d_vec)
                    dv[pl.ds(g * L, L)] = d_vec
                    return gcarry

                lax.fori_loop(0, C // L, group_body, 0)
                pltpu.sync_copy(he, h_out.at[pl.ds(base, C)])
                pltpu.sync_copy(te, t_out.at[pl.ds(base, C)])
                pltpu.sync_copy(dv, dist_o.at[pl.ds(base, C)])
                return carry

            lax.fori_loop(0, n_chunks, chunk_body, 0)

        do_side(ph, pt, pr, pos_o, phe_o, pte_o)
        do_side(nh, nt, nr, neg_o, nhe_o, nte_o)

    return k(pos_h, pos_t, pos_r, neg_h, neg_t, neg_r,
             ent_emb, rel_emb, ent_proj, rel_proj)


def kernel(pos_h, pos_t, pos_r, neg_h, neg_t, neg_r,
           ent_emb, rel_emb, ent_proj, rel_proj):
    i32 = jnp.int32
    d_real = ent_emb.shape[1]
    pad = ((0, 0), (0, 128 - d_real))
    pos, neg, phe, pte, nhe, nte = _dynmap_sc(
        pos_h.astype(i32), pos_t.astype(i32), pos_r.astype(i32),
        neg_h.astype(i32), neg_t.astype(i32), neg_r.astype(i32),
        jnp.pad(ent_emb, pad), jnp.pad(rel_emb, pad),
        jnp.pad(ent_proj, pad), jnp.pad(rel_proj, pad), d_real)
    return (pos, neg, phe[:, :d_real], pte[:, :d_real],
            nhe[:, :d_real], nte[:, :d_real])


# speedup vs baseline: 1.7535x; 1.6592x over previous
import functools

import jax
import jax.numpy as jnp
from jax import lax
from jax.experimental import pallas as pl
from jax.experimental.pallas import tpu as pltpu
from jax.experimental.pallas import tpu_sc as plsc

def _retile_body(t1_ref, t2_ref, o_ref):
    o_ref[:, 0:64] = jnp.transpose(t1_ref[...])
    o_ref[:, 64:128] = jnp.transpose(t2_ref[...])


def _retile(t, p, bc):
    n = t.shape[1]
    grid = p // bc
    max_blk = (n - 1) // bc

    return pl.pallas_call(
        _retile_body,
        grid=(grid,),
        in_specs=[
            pl.BlockSpec((64, bc), lambda i: (0, i)),
            pl.BlockSpec(
                (64, bc),
                lambda i, g=grid, m=max_blk: (0, jnp.minimum(i + g, m)),
            ),
        ],
        out_specs=pl.BlockSpec((bc, 128), lambda i: (i, 0)),
        out_shape=jax.ShapeDtypeStruct((p, 128), jnp.float32),
    )(t, t)


def _dynmap_sc(ph2, php, pt2, ptp, pr2, prp, nh2, nhp, nt2, ntp, nr2, nrp,
               ee, rel_e, ep, rel_p, d_real):
    B = ph2.shape[0]
    D = 128
    info = plsc.get_sparse_core_info()
    NC, NS, L = info.num_cores, info.num_subcores, info.num_lanes
    NW = NC * NS
    C = 128
    b_per_w = B // NW
    n_chunks = b_per_w // C
    KV = d_real // L

    mesh = plsc.VectorSubcoreMesh(core_axis_name="c", subcore_axis_name="s")
    f32 = jnp.float32
    out_type = (
        jax.ShapeDtypeStruct((B,), f32),
        jax.ShapeDtypeStruct((B,), f32),
        jax.ShapeDtypeStruct((B, D), f32),
        jax.ShapeDtypeStruct((B, D), f32),
        jax.ShapeDtypeStruct((B, D), f32),
        jax.ShapeDtypeStruct((B, D), f32),
    )

    @functools.partial(
        pl.kernel,
        mesh=mesh,
        out_type=out_type,
        compiler_params=pltpu.CompilerParams(needs_layout_passes=False),
        scratch_types=[
            pltpu.VMEM((C,), jnp.int32),
            pltpu.VMEM((C,), jnp.int32),
            pltpu.VMEM((C,), jnp.int32),
            pltpu.VMEM((C,), jnp.int32),
            pltpu.VMEM((C,), jnp.int32),
            pltpu.VMEM((C,), jnp.int32),
            pltpu.VMEM((C, D), f32),
            pltpu.VMEM((C, D), f32),
            pltpu.VMEM((C, D), f32),
            pltpu.VMEM((C, D), f32),
            pltpu.VMEM((C, D), f32),
            pltpu.VMEM((C, D), f32),
            pltpu.VMEM((C,), f32),
            pltpu.SemaphoreType.DMA,
        ],
    )
    def k(phi, phpar, pti, ptpar, pri, prpar, nhi, nhpar, nti, ntpar,
          nri, nrpar, eet, rele, ept, relp,
          pos_o, neg_o, phe_o, pte_o, nhe_o, nte_o,
          hi, ti, ri, hpv_, tpv_, rpv_, he, hp, te, tp, re_v, rp_v,
          dv, sem):
        wid = lax.axis_index("s") * NC + lax.axis_index("c")
        lane = lax.iota(jnp.int32, L)

        def do_side(h_idx, h_par, t_idx, t_par, r_idx, r_par,
                    dist_o, h_out, t_out):
            def chunk_body(c, carry):
                base = wid * b_per_w + c * C
                pltpu.sync_copy(h_idx.at[pl.ds(base, C)], hi)
                pltpu.sync_copy(t_idx.at[pl.ds(base, C)], ti)
                pltpu.sync_copy(r_idx.at[pl.ds(base, C)], ri)
                pltpu.sync_copy(h_par.at[pl.ds(base, C)], hpv_)
                pltpu.sync_copy(t_par.at[pl.ds(base, C)], tpv_)
                pltpu.sync_copy(r_par.at[pl.ds(base, C)], rpv_)
                cps = [
                    pltpu.async_copy(eet.at[hi], he, sem),
                    pltpu.async_copy(ept.at[hi], hp, sem),
                    pltpu.async_copy(eet.at[ti], te, sem),
                    pltpu.async_copy(ept.at[ti], tp, sem),
                    pltpu.async_copy(rele.at[ri], re_v, sem),
                    pltpu.async_copy(relp.at[ri], rp_v, sem),
                ]
                for cp in cps:
                    cp.wait()

                def group_body(g, gcarry):
                    d_vec = jnp.zeros((L,), f32)
                    hoff_v = hpv_[pl.ds(g * L, L)]
                    toff_v = tpv_[pl.ds(g * L, L)]
                    roff_v = rpv_[pl.ds(g * L, L)]
                    for rl in range(L):
                        row = g * L + rl
                        hb = hoff_v[rl]
                        tb = toff_v[rl]
                        rb = roff_v[rl]
                        hv = [he[row, pl.ds(hb + kk * L, L)] for kk in range(KV)]
                        hpv = [hp[row, pl.ds(hb + kk * L, L)] for kk in range(KV)]
                        tv = [te[row, pl.ds(tb + kk * L, L)] for kk in range(KV)]
                        tpv = [tp[row, pl.ds(tb + kk * L, L)] for kk in range(KV)]
                        rv = [re_v[row, pl.ds(rb + kk * L, L)] for kk in range(KV)]
                        rpv = [rp_v[row, pl.ds(rb + kk * L, L)] for kk in range(KV)]
                        hdot = hv[0] * hpv[0]
                        tdot = tv[0] * tpv[0]
                        for kk in range(1, KV):
                            hdot = hdot + hv[kk] * hpv[kk]
                            tdot = tdot + tv[kk] * tpv[kk]
                        sh = jnp.sum(hdot)
                        st = jnp.sum(tdot)
                        dacc = None
                        for kk in range(KV):
                            hprime = hv[kk] + sh * rpv[kk]
                            tprime = tv[kk] + st * rpv[kk]
                            he[row, pl.ds(kk * L, L)] = hprime
                            te[row, pl.ds(kk * L, L)] = tprime
                            term = jnp.abs(hprime + rv[kk] - tprime)
                            dacc = term if dacc is None else dacc + term
                        d = jnp.sum(dacc)
                        d_vec = jnp.where(lane == rl, d, d_vec)
                    dv[pl.ds(g * L, L)] = d_vec
                    return gcarry

                lax.fori_loop(0, C // L, group_body, 0)
                pltpu.sync_copy(he, h_out.at[pl.ds(base, C)])
                pltpu.sync_copy(te, t_out.at[pl.ds(base, C)])
                pltpu.sync_copy(dv, dist_o.at[pl.ds(base, C)])
                return carry

            lax.fori_loop(0, n_chunks, chunk_body, 0)

        do_side(phi, phpar, pti, ptpar, pri, prpar, pos_o, phe_o, pte_o)
        do_side(nhi, nhpar, nti, ntpar, nri, nrpar, neg_o, nhe_o, nte_o)

    return k(ph2, php, pt2, ptp, pr2, prp, nh2, nhp, nt2, ntp, nr2, nrp,
             ee, rel_e, ep, rel_p)


def kernel(pos_h, pos_t, pos_r, neg_h, neg_t, neg_r,
           ent_emb, rel_emb, ent_proj, rel_proj):
    i32 = jnp.int32
    d_real = ent_emb.shape[1]
    e_p = 524288
    r_p = 512

    r_pad = ((0, 2 * r_p - rel_emb.shape[0]), (0, 0))
    ee2 = _retile(ent_emb.T, e_p, 4096)
    ep2 = _retile(ent_proj.T, e_p, 4096)
    re2 = _retile(jnp.pad(rel_emb, r_pad).T, r_p, r_p)
    rp2 = _retile(jnp.pad(rel_proj, r_pad).T, r_p, r_p)

    def split(x, p):
        x = x.astype(i32)
        lo = x < p
        return jnp.where(lo, x, x - p), jnp.where(lo, 0, d_real)

    ph2, php = split(pos_h, e_p)
    pt2, ptp = split(pos_t, e_p)
    pr2, prp = split(pos_r, r_p)
    nh2, nhp = split(neg_h, e_p)
    nt2, ntp = split(neg_t, e_p)
    nr2, nrp = split(neg_r, r_p)

    pos, neg, phe, pte, nhe, nte = _dynmap_sc(
        ph2, php, pt2, ptp, pr2, prp, nh2, nhp, nt2, ntp, nr2, nrp,
        ee2, re2, ep2, rp2, d_real)
    return (pos, neg, phe[:, :d_real], pte[:, :d_real],
            nhe[:, :d_real], nte[:, :d_real])


# speedup vs baseline: 2.1581x; 1.2308x over previous
import functools

import jax
import jax.numpy as jnp
from jax import lax
from jax.experimental import pallas as pl
from jax.experimental.pallas import tpu as pltpu
from jax.experimental.pallas import tpu_sc as plsc

def _retile_body(t1_ref, t2_ref, o_ref):
    z = jnp.concatenate([t1_ref[...], t2_ref[...]], axis=0)
    o_ref[...] = jnp.transpose(z)


def _retile(t, p, bc):
    n = t.shape[1]
    grid = p // bc
    max_blk = (n - 1) // bc

    return pl.pallas_call(
        _retile_body,
        grid=(grid,),
        in_specs=[
            pl.BlockSpec((64, bc), lambda i: (0, i)),
            pl.BlockSpec(
                (64, bc),
                lambda i, g=grid, m=max_blk: (0, jnp.minimum(i + g, m)),
            ),
        ],
        out_specs=pl.BlockSpec((bc, 128), lambda i: (i, 0)),
        out_shape=jax.ShapeDtypeStruct((p, 128), jnp.float32),
    )(t, t)


def _dynmap_sc(ph2, php, pt2, ptp, pr2, prp, nh2, nhp, nt2, ntp, nr2, nrp,
               ee, rel_e, ep, rel_p, d_real):
    B = ph2.shape[0]
    D = 128
    info = plsc.get_sparse_core_info()
    NC, NS, L = info.num_cores, info.num_subcores, info.num_lanes
    NW = NC * NS
    C = 128
    b_per_w = B // NW
    n_chunks = b_per_w // C
    KV = d_real // L

    mesh = plsc.VectorSubcoreMesh(core_axis_name="c", subcore_axis_name="s")
    f32 = jnp.float32
    out_type = (
        jax.ShapeDtypeStruct((B,), f32),
        jax.ShapeDtypeStruct((B,), f32),
        jax.ShapeDtypeStruct((B, D), f32),
        jax.ShapeDtypeStruct((B, D), f32),
        jax.ShapeDtypeStruct((B, D), f32),
        jax.ShapeDtypeStruct((B, D), f32),
    )

    @functools.partial(
        pl.kernel,
        mesh=mesh,
        out_type=out_type,
        compiler_params=pltpu.CompilerParams(needs_layout_passes=False),
        scratch_types=[
            pltpu.VMEM((C,), jnp.int32),
            pltpu.VMEM((C,), jnp.int32),
            pltpu.VMEM((C,), jnp.int32),
            pltpu.VMEM((C,), jnp.int32),
            pltpu.VMEM((C,), jnp.int32),
            pltpu.VMEM((C,), jnp.int32),
            pltpu.VMEM((C, D), f32),
            pltpu.VMEM((C, D), f32),
            pltpu.VMEM((C, D), f32),
            pltpu.VMEM((C, D), f32),
            pltpu.VMEM((C, D), f32),
            pltpu.VMEM((C, D), f32),
            pltpu.VMEM((C,), f32),
            pltpu.SemaphoreType.DMA,
        ],
    )
    def k(phi, phpar, pti, ptpar, pri, prpar, nhi, nhpar, nti, ntpar,
          nri, nrpar, eet, rele, ept, relp,
          pos_o, neg_o, phe_o, pte_o, nhe_o, nte_o,
          hi, ti, ri, hpv_, tpv_, rpv_, he, hp, te, tp, re_v, rp_v,
          dv, sem):
        wid = lax.axis_index("s") * NC + lax.axis_index("c")
        lane = lax.iota(jnp.int32, L)

        def do_side(h_idx, h_par, t_idx, t_par, r_idx, r_par,
                    dist_o, h_out, t_out):
            def chunk_body(c, carry):
                base = wid * b_per_w + c * C
                pltpu.sync_copy(h_idx.at[pl.ds(base, C)], hi)
                pltpu.sync_copy(t_idx.at[pl.ds(base, C)], ti)
                pltpu.sync_copy(r_idx.at[pl.ds(base, C)], ri)
                pltpu.sync_copy(h_par.at[pl.ds(base, C)], hpv_)
                pltpu.sync_copy(t_par.at[pl.ds(base, C)], tpv_)
                pltpu.sync_copy(r_par.at[pl.ds(base, C)], rpv_)
                cps = [
                    pltpu.async_copy(eet.at[hi], he, sem),
                    pltpu.async_copy(ept.at[hi], hp, sem),
                    pltpu.async_copy(eet.at[ti], te, sem),
                    pltpu.async_copy(ept.at[ti], tp, sem),
                    pltpu.async_copy(rele.at[ri], re_v, sem),
                    pltpu.async_copy(relp.at[ri], rp_v, sem),
                ]
                for cp in cps:
                    cp.wait()

                def group_body(g, gcarry):
                    d_vec = jnp.zeros((L,), f32)
                    hoff_v = hpv_[pl.ds(g * L, L)]
                    toff_v = tpv_[pl.ds(g * L, L)]
                    roff_v = rpv_[pl.ds(g * L, L)]
                    for rl in range(L):
                        row = g * L + rl
                        hb = hoff_v[rl]
                        tb = toff_v[rl]
                        rb = roff_v[rl]
                        hv = [he[row, pl.ds(hb + kk * L, L)] for kk in range(KV)]
                        hpv = [hp[row, pl.ds(hb + kk * L, L)] for kk in range(KV)]
                        tv = [te[row, pl.ds(tb + kk * L, L)] for kk in range(KV)]
                        tpv = [tp[row, pl.ds(tb + kk * L, L)] for kk in range(KV)]
                        rv = [re_v[row, pl.ds(rb + kk * L, L)] for kk in range(KV)]
                        rpv = [rp_v[row, pl.ds(rb + kk * L, L)] for kk in range(KV)]
                        hdot = hv[0] * hpv[0]
                        tdot = tv[0] * tpv[0]
                        for kk in range(1, KV):
                            hdot = hdot + hv[kk] * hpv[kk]
                            tdot = tdot + tv[kk] * tpv[kk]
                        sh = jnp.sum(hdot)
                        st = jnp.sum(tdot)
                        dacc = None
                        for kk in range(KV):
                            hprime = hv[kk] + sh * rpv[kk]
                            tprime = tv[kk] + st * rpv[kk]
                            he[row, pl.ds(kk * L, L)] = hprime
                            te[row, pl.ds(kk * L, L)] = tprime
                            term = jnp.abs(hprime + rv[kk] - tprime)
                            dacc = term if dacc is None else dacc + term
                        d = jnp.sum(dacc)
                        d_vec = jnp.where(lane == rl, d, d_vec)
                    dv[pl.ds(g * L, L)] = d_vec
                    return gcarry

                lax.fori_loop(0, C // L, group_body, 0)
                pltpu.sync_copy(he, h_out.at[pl.ds(base, C)])
                pltpu.sync_copy(te, t_out.at[pl.ds(base, C)])
                pltpu.sync_copy(dv, dist_o.at[pl.ds(base, C)])
                return carry

            lax.fori_loop(0, n_chunks, chunk_body, 0)

        do_side(phi, phpar, pti, ptpar, pri, prpar, pos_o, phe_o, pte_o)
        do_side(nhi, nhpar, nti, ntpar, nri, nrpar, neg_o, nhe_o, nte_o)

    return k(ph2, php, pt2, ptp, pr2, prp, nh2, nhp, nt2, ntp, nr2, nrp,
             ee, rel_e, ep, rel_p)


def kernel(pos_h, pos_t, pos_r, neg_h, neg_t, neg_r,
           ent_emb, rel_emb, ent_proj, rel_proj):
    i32 = jnp.int32
    d_real = ent_emb.shape[1]
    e_p = 524288
    r_p = 512

    r_pad = ((0, 2 * r_p - rel_emb.shape[0]), (0, 0))
    ee2 = _retile(ent_emb.T, e_p, 4096)
    ep2 = _retile(ent_proj.T, e_p, 4096)
    re2 = _retile(jnp.pad(rel_emb, r_pad).T, r_p, r_p)
    rp2 = _retile(jnp.pad(rel_proj, r_pad).T, r_p, r_p)

    def split(x, p):
        x = x.astype(i32)
        lo = x < p
        return jnp.where(lo, x, x - p), jnp.where(lo, 0, d_real)

    ph2, php = split(pos_h, e_p)
    pt2, ptp = split(pos_t, e_p)
    pr2, prp = split(pos_r, r_p)
    nh2, nhp = split(neg_h, e_p)
    nt2, ntp = split(neg_t, e_p)
    nr2, nrp = split(neg_r, r_p)

    pos, neg, phe, pte, nhe, nte = _dynmap_sc(
        ph2, php, pt2, ptp, pr2, prp, nh2, nhp, nt2, ntp, nr2, nrp,
        ee2, re2, ep2, rp2, d_real)
    return (pos, neg, phe[:, :d_real], pte[:, :d_real],
            nhe[:, :d_real], nte[:, :d_real])


# speedup vs baseline: 2.4188x; 1.1208x over previous
import functools

import jax
import jax.numpy as jnp
from jax import lax
from jax.experimental import pallas as pl
from jax.experimental.pallas import tpu as pltpu
from jax.experimental.pallas import tpu_sc as plsc

def _retile_body(t1_ref, t2_ref, o_ref):
    z = jnp.concatenate([t1_ref[...], t2_ref[...]], axis=0)
    o_ref[...] = jnp.transpose(z)


def _retile(t, p, bc):
    n = t.shape[1]
    grid = p // bc
    max_blk = (n - 1) // bc

    return pl.pallas_call(
        _retile_body,
        grid=(grid,),
        in_specs=[
            pl.BlockSpec((64, bc), lambda i: (0, i)),
            pl.BlockSpec(
                (64, bc),
                lambda i, g=grid, m=max_blk: (0, jnp.minimum(i + g, m)),
            ),
        ],
        out_specs=pl.BlockSpec((bc, 128), lambda i: (i, 0)),
        out_shape=jax.ShapeDtypeStruct((p, 128), jnp.float32),
    )(t, t)


def _dynmap_sc(ph2, php, pt2, ptp, pr2, prp, nh2, nhp, nt2, ntp, nr2, nrp,
               ee, rel_e, ep, rel_p, d_real):
    B = ph2.shape[0]
    D = 128
    info = plsc.get_sparse_core_info()
    NC, NS, L = info.num_cores, info.num_subcores, info.num_lanes
    NW = NC * NS
    C = 128
    b_per_w = B // NW
    n_chunks = b_per_w // C
    KV = d_real // L

    mesh = plsc.VectorSubcoreMesh(core_axis_name="c", subcore_axis_name="s")
    f32 = jnp.float32
    out_type = (
        jax.ShapeDtypeStruct((B,), f32),
        jax.ShapeDtypeStruct((B,), f32),
        jax.ShapeDtypeStruct((B, D), f32),
        jax.ShapeDtypeStruct((B, D), f32),
        jax.ShapeDtypeStruct((B, D), f32),
        jax.ShapeDtypeStruct((B, D), f32),
    )

    @functools.partial(
        pl.kernel,
        mesh=mesh,
        out_type=out_type,
        compiler_params=pltpu.CompilerParams(needs_layout_passes=False),
        scratch_types=[
            pltpu.VMEM((C,), jnp.int32),
            pltpu.VMEM((C,), jnp.int32),
            pltpu.VMEM((C,), jnp.int32),
            pltpu.VMEM((C,), jnp.int32),
            pltpu.VMEM((C,), jnp.int32),
            pltpu.VMEM((C,), jnp.int32),
            pltpu.VMEM((C, D), f32),
            pltpu.VMEM((C, D), f32),
            pltpu.VMEM((C, D), f32),
            pltpu.VMEM((C, D), f32),
            pltpu.VMEM((C, D), f32),
            pltpu.VMEM((C, D), f32),
            pltpu.VMEM((C,), f32),
            pltpu.SemaphoreType.DMA,
        ],
    )
    def k(phi, phpar, pti, ptpar, pri, prpar, nhi, nhpar, nti, ntpar,
          nri, nrpar, eet, rele, ept, relp,
          pos_o, neg_o, phe_o, pte_o, nhe_o, nte_o,
          hi, ti, ri, hpv_, tpv_, rpv_, he, hp, te, tp, re_v, rp_v,
          dv, sem):
        wid = lax.axis_index("s") * NC + lax.axis_index("c")
        lane = lax.iota(jnp.int32, L)

        def do_side(h_idx, h_par, t_idx, t_par, r_idx, r_par,
                    dist_o, h_out, t_out):
            def chunk_body(c, carry):
                base = wid * b_per_w + c * C
                pltpu.sync_copy(h_idx.at[pl.ds(base, C)], hi)
                pltpu.sync_copy(t_idx.at[pl.ds(base, C)], ti)
                pltpu.sync_copy(r_idx.at[pl.ds(base, C)], ri)
                pltpu.sync_copy(h_par.at[pl.ds(base, C)], hpv_)
                pltpu.sync_copy(t_par.at[pl.ds(base, C)], tpv_)
                pltpu.sync_copy(r_par.at[pl.ds(base, C)], rpv_)
                cps = [
                    pltpu.async_copy(eet.at[hi], he, sem),
                    pltpu.async_copy(ept.at[hi], hp, sem),
                    pltpu.async_copy(eet.at[ti], te, sem),
                    pltpu.async_copy(ept.at[ti], tp, sem),
                    pltpu.async_copy(rele.at[ri], re_v, sem),
                    pltpu.async_copy(relp.at[ri], rp_v, sem),
                ]
                for cp in cps:
                    cp.wait()

                def group_body(g, gcarry):
                    d_vec = jnp.zeros((L,), f32)
                    hoff_v = hpv_[pl.ds(g * L, L)]
                    toff_v = tpv_[pl.ds(g * L, L)]
                    roff_v = rpv_[pl.ds(g * L, L)]
                    for rl in range(L):
                        row = g * L + rl
                        hb = hoff_v[rl]
                        tb = toff_v[rl]
                        rb = roff_v[rl]
                        hv = [he[row, pl.ds(hb + kk * L, L)] for kk in range(KV)]
                        hpv = [hp[row, pl.ds(hb + kk * L, L)] for kk in range(KV)]
                        tv = [te[row, pl.ds(tb + kk * L, L)] for kk in range(KV)]
                        tpv = [tp[row, pl.ds(tb + kk * L, L)] for kk in range(KV)]
                        rv = [re_v[row, pl.ds(rb + kk * L, L)] for kk in range(KV)]
                        rpv = [rp_v[row, pl.ds(rb + kk * L, L)] for kk in range(KV)]
                        hdot = hv[0] * hpv[0]
                        tdot = tv[0] * tpv[0]
                        for kk in range(1, KV):
                            hdot = hdot + hv[kk] * hpv[kk]
                            tdot = tdot + tv[kk] * tpv[kk]
                        sh = jnp.sum(hdot)
                        st = jnp.sum(tdot)
                        dacc = None
                        for kk in range(KV):
                            hprime = hv[kk] + sh * rpv[kk]
                            tprime = tv[kk] + st * rpv[kk]
                            he[row, pl.ds(kk * L, L)] = hprime
                            te[row, pl.ds(kk * L, L)] = tprime
                            term = jnp.abs(hprime + rv[kk] - tprime)
                            dacc = term if dacc is None else dacc + term
                        d = jnp.sum(dacc)
                        d_vec = jnp.where(lane == rl, d, d_vec)
                    dv[pl.ds(g * L, L)] = d_vec
                    return gcarry

                lax.fori_loop(0, C // L, group_body, 0)
                pltpu.sync_copy(he, h_out.at[pl.ds(base, C)])
                pltpu.sync_copy(te, t_out.at[pl.ds(base, C)])
                pltpu.sync_copy(dv, dist_o.at[pl.ds(base, C)])
                return carry

            lax.fori_loop(0, n_chunks, chunk_body, 0)

        do_side(phi, phpar, pti, ptpar, pri, prpar, pos_o, phe_o, pte_o)
        do_side(nhi, nhpar, nti, ntpar, nri, nrpar, neg_o, nhe_o, nte_o)

    return k(ph2, php, pt2, ptp, pr2, prp, nh2, nhp, nt2, ntp, nr2, nrp,
             ee, rel_e, ep, rel_p)


def kernel(pos_h, pos_t, pos_r, neg_h, neg_t, neg_r,
           ent_emb, rel_emb, ent_proj, rel_proj):
    i32 = jnp.int32
    d_real = ent_emb.shape[1]
    e_p = 524288
    r_p = 512

    r_pad = ((0, 2 * r_p - rel_emb.shape[0]), (0, 0))
    ee2 = _retile(ent_emb.T, e_p, 8192)
    ep2 = _retile(ent_proj.T, e_p, 8192)
    re2 = _retile(jnp.pad(rel_emb, r_pad).T, r_p, r_p)
    rp2 = _retile(jnp.pad(rel_proj, r_pad).T, r_p, r_p)

    def split(x, p):
        x = x.astype(i32)
        lo = x < p
        return jnp.where(lo, x, x - p), jnp.where(lo, 0, d_real)

    ph2, php = split(pos_h, e_p)
    pt2, ptp = split(pos_t, e_p)
    pr2, prp = split(pos_r, r_p)
    nh2, nhp = split(neg_h, e_p)
    nt2, ntp = split(neg_t, e_p)
    nr2, nrp = split(neg_r, r_p)

    pos, neg, phe, pte, nhe, nte = _dynmap_sc(
        ph2, php, pt2, ptp, pr2, prp, nh2, nhp, nt2, ntp, nr2, nrp,
        ee2, re2, ep2, rp2, d_real)
    return (pos, neg, phe[:, :d_real], pte[:, :d_real],
            nhe[:, :d_real], nte[:, :d_real])


# speedup vs baseline: 2.4746x; 1.0231x over previous
import functools

import jax
import jax.numpy as jnp
from jax import lax
from jax.experimental import pallas as pl
from jax.experimental.pallas import tpu as pltpu
from jax.experimental.pallas import tpu_sc as plsc

def _retile_body(t1_ref, t2_ref, o_ref):
    z = jnp.concatenate([t1_ref[...], t2_ref[...]], axis=0)
    o_ref[...] = jnp.transpose(z)


def _retile(t, p, bc):
    n = t.shape[1]
    grid = p // bc
    max_blk = (n - 1) // bc

    return pl.pallas_call(
        _retile_body,
        grid=(grid,),
        in_specs=[
            pl.BlockSpec((64, bc), lambda i: (0, i)),
            pl.BlockSpec(
                (64, bc),
                lambda i, g=grid, m=max_blk: (0, jnp.minimum(i + g, m)),
            ),
        ],
        out_specs=pl.BlockSpec((bc, 128), lambda i: (i, 0)),
        out_shape=jax.ShapeDtypeStruct((p, 128), jnp.float32),
    )(t, t)


def _dynmap_sc(ph2, php, pt2, ptp, pr2, prp, nh2, nhp, nt2, ntp, nr2, nrp,
               ee, rel_e, ep, rel_p, d_real):
    B = ph2.shape[0]
    D = 128
    info = plsc.get_sparse_core_info()
    NC, NS, L = info.num_cores, info.num_subcores, info.num_lanes
    NW = NC * NS
    C = 128
    b_per_w = B // NW
    n_chunks = b_per_w // C
    KV = d_real // L

    mesh = plsc.VectorSubcoreMesh(core_axis_name="c", subcore_axis_name="s")
    f32 = jnp.float32
    out_type = (
        jax.ShapeDtypeStruct((B,), f32),
        jax.ShapeDtypeStruct((B,), f32),
        jax.ShapeDtypeStruct((B, D), f32),
        jax.ShapeDtypeStruct((B, D), f32),
        jax.ShapeDtypeStruct((B, D), f32),
        jax.ShapeDtypeStruct((B, D), f32),
    )

    @functools.partial(
        pl.kernel,
        mesh=mesh,
        out_type=out_type,
        compiler_params=pltpu.CompilerParams(needs_layout_passes=False),
        scratch_types=[
            pltpu.VMEM((C,), jnp.int32),
            pltpu.VMEM((C,), jnp.int32),
            pltpu.VMEM((C,), jnp.int32),
            pltpu.VMEM((C,), jnp.int32),
            pltpu.VMEM((C,), jnp.int32),
            pltpu.VMEM((C,), jnp.int32),
            pltpu.VMEM((C, D), f32),
            pltpu.VMEM((C, D), f32),
            pltpu.VMEM((C, D), f32),
            pltpu.VMEM((C, D), f32),
            pltpu.VMEM((C, D), f32),
            pltpu.VMEM((C, D), f32),
            pltpu.VMEM((C,), f32),
            pltpu.SemaphoreType.DMA,
        ],
    )
    def k(phi, phpar, pti, ptpar, pri, prpar, nhi, nhpar, nti, ntpar,
          nri, nrpar, eet, rele, ept, relp,
          pos_o, neg_o, phe_o, pte_o, nhe_o, nte_o,
          hi, ti, ri, hpv_, tpv_, rpv_, he, hp, te, tp, re_v, rp_v,
          dv, sem):
        wid = lax.axis_index("s") * NC + lax.axis_index("c")
        lane = lax.iota(jnp.int32, L)

        def do_side(h_idx, h_par, t_idx, t_par, r_idx, r_par,
                    dist_o, h_out, t_out):
            def chunk_body(c, carry):
                base = wid * b_per_w + c * C
                pltpu.sync_copy(h_idx.at[pl.ds(base, C)], hi)
                pltpu.sync_copy(t_idx.at[pl.ds(base, C)], ti)
                pltpu.sync_copy(r_idx.at[pl.ds(base, C)], ri)
                pltpu.sync_copy(h_par.at[pl.ds(base, C)], hpv_)
                pltpu.sync_copy(t_par.at[pl.ds(base, C)], tpv_)
                pltpu.sync_copy(r_par.at[pl.ds(base, C)], rpv_)
                cps = [
                    pltpu.async_copy(eet.at[hi], he, sem),
                    pltpu.async_copy(ept.at[hi], hp, sem),
                    pltpu.async_copy(eet.at[ti], te, sem),
                    pltpu.async_copy(ept.at[ti], tp, sem),
                    pltpu.async_copy(rele.at[ri], re_v, sem),
                    pltpu.async_copy(relp.at[ri], rp_v, sem),
                ]
                for cp in cps:
                    cp.wait()

                def group_body(g, gcarry):
                    d_vec = jnp.zeros((L,), f32)
                    hoff_v = hpv_[pl.ds(g * L, L)]
                    toff_v = tpv_[pl.ds(g * L, L)]
                    roff_v = rpv_[pl.ds(g * L, L)]
                    for rl in range(L):
                        row = g * L + rl
                        hb = hoff_v[rl]
                        tb = toff_v[rl]
                        rb = roff_v[rl]
                        hv = [he[row, pl.ds(hb + kk * L, L)] for kk in range(KV)]
                        hpv = [hp[row, pl.ds(hb + kk * L, L)] for kk in range(KV)]
                        tv = [te[row, pl.ds(tb + kk * L, L)] for kk in range(KV)]
                        tpv = [tp[row, pl.ds(tb + kk * L, L)] for kk in range(KV)]
                        rv = [re_v[row, pl.ds(rb + kk * L, L)] for kk in range(KV)]
                        rpv = [rp_v[row, pl.ds(rb + kk * L, L)] for kk in range(KV)]
                        hdot = hv[0] * hpv[0]
                        tdot = tv[0] * tpv[0]
                        for kk in range(1, KV):
                            hdot = hdot + hv[kk] * hpv[kk]
                            tdot = tdot + tv[kk] * tpv[kk]
                        sh = jnp.sum(hdot)
                        st = jnp.sum(tdot)
                        dacc = None
                        for kk in range(KV):
                            hprime = hv[kk] + sh * rpv[kk]
                            tprime = tv[kk] + st * rpv[kk]
                            he[row, pl.ds(kk * L, L)] = hprime
                            te[row, pl.ds(kk * L, L)] = tprime
                            term = jnp.abs(hprime + rv[kk] - tprime)
                            dacc = term if dacc is None else dacc + term
                        d = jnp.sum(dacc)
                        d_vec = jnp.where(lane == rl, d, d_vec)
                    dv[pl.ds(g * L, L)] = d_vec
                    return gcarry

                lax.fori_loop(0, C // L, group_body, 0)
                pltpu.sync_copy(he, h_out.at[pl.ds(base, C)])
                pltpu.sync_copy(te, t_out.at[pl.ds(base, C)])
                pltpu.sync_copy(dv, dist_o.at[pl.ds(base, C)])
                return carry

            lax.fori_loop(0, n_chunks, chunk_body, 0)

        do_side(phi, phpar, pti, ptpar, pri, prpar, pos_o, phe_o, pte_o)
        do_side(nhi, nhpar, nti, ntpar, nri, nrpar, neg_o, nhe_o, nte_o)

    return k(ph2, php, pt2, ptp, pr2, prp, nh2, nhp, nt2, ntp, nr2, nrp,
             ee, rel_e, ep, rel_p)


def kernel(pos_h, pos_t, pos_r, neg_h, neg_t, neg_r,
           ent_emb, rel_emb, ent_proj, rel_proj):
    i32 = jnp.int32
    d_real = ent_emb.shape[1]
    e_p = 524288
    r_p = 512

    r_pad = ((0, 2 * r_p - rel_emb.shape[0]), (0, 0))
    ee2 = _retile(ent_emb.T, e_p, 16384)
    ep2 = _retile(ent_proj.T, e_p, 16384)
    re2 = _retile(jnp.pad(rel_emb, r_pad).T, r_p, r_p)
    rp2 = _retile(jnp.pad(rel_proj, r_pad).T, r_p, r_p)

    def split(x, p):
        x = x.astype(i32)
        lo = x < p
        return jnp.where(lo, x, x - p), jnp.where(lo, 0, d_real)

    ph2, php = split(pos_h, e_p)
    pt2, ptp = split(pos_t, e_p)
    pr2, prp = split(pos_r, r_p)
    nh2, nhp = split(neg_h, e_p)
    nt2, ntp = split(neg_t, e_p)
    nr2, nrp = split(neg_r, r_p)

    pos, neg, phe, pte, nhe, nte = _dynmap_sc(
        ph2, php, pt2, ptp, pr2, prp, nh2, nhp, nt2, ntp, nr2, nrp,
        ee2, re2, ep2, rp2, d_real)
    return (pos, neg, phe[:, :d_real], pte[:, :d_real],
            nhe[:, :d_real], nte[:, :d_real])
